# Initial kernel scaffold; baseline (speedup 1.0000x reference)
import functools

import jax
import jax.numpy as jnp
from jax import lax
from jax.experimental import pallas as pl
from jax.experimental.pallas import tpu as pltpu
from jax.experimental.pallas import tpu_sc as plsc

N = 100000
E = 1600000
IN_DIM = 26
HID = 32
OUT = 32
G = 2048

NC, NS = 2, 16
NPAD = 102400
EPAD = 1605632
EROWS = EPAD // 128
GP = G + 8

_ETILE = EROWS // NS
_GRP = 49
_NGRP = _ETILE // _GRP
_DROWS = EROWS // (NC * NS)
_BROWS = (NPAD // 128) // (NC * NS)
_NT = NPAD // NS


def _mesh():
    return plsc.VectorSubcoreMesh(core_axis_name="c", subcore_axis_name="s",
                                  num_cores=NC, num_subcores=NS)


def _fill(ref, val):
    def body(i, _):
        ref[i] = jnp.full((16,), val, jnp.float32)
        return 0
    lax.fori_loop(0, 128, body, 0)


def _zero_rows(dst, base, n, zer_v, sem):
    ds = [pltpu.async_copy(zer_v, dst.at[pl.ds(base + i * 128, 128)], sem)
          for i in range(n)]
    for d in ds:
        d.wait()


@functools.partial(
    pl.kernel,
    out_type=[jax.ShapeDtypeStruct((NC, NPAD, 16), jnp.float32),
              jax.ShapeDtypeStruct((NC, GP, 16), jnp.float32)],
    mesh=_mesh(),
    scratch_types=[
        pltpu.VMEM_SHARED((NPAD, 16), jnp.float32),
        pltpu.VMEM_SHARED((GP, 16), jnp.float32),
        pltpu.VMEM((_DROWS, 128), jnp.int32),
        pltpu.VMEM((128, 16), jnp.float32),
        pltpu.VMEM((128, 16), jnp.float32),
        pltpu.SemaphoreType.DMA,
    ],
)
def _prep(dst2d, batch2d, degp, cntp, deg_s, cnt_s, idx_v, ones_v, zer_v, sem):
    c = lax.axis_index("c")
    s = lax.axis_index("s")
    _fill(ones_v, 1.0)
    _fill(zer_v, 0.0)
    _zero_rows(deg_s, s * _NT, _NT // 128, zer_v, sem)

    @pl.when(s == 0)
    def _():
        _zero_rows(cnt_s, 0, G // 128, zer_v, sem)
        pltpu.sync_copy(zer_v.at[pl.ds(0, 8)], cnt_s.at[pl.ds(G, 8)])

    plsc.subcore_barrier()

    row0 = c * (EROWS // NC) + s * _DROWS
    pltpu.sync_copy(dst2d.at[pl.ds(row0, _DROWS)], idx_v)

    def grp(gi, _):
        ds = [pltpu.async_copy(ones_v, deg_s.at[idx_v.at[gi * _GRP + j]],
                               sem, add=True) for j in range(_GRP)]
        for d in ds:
            d.wait()
        return 0
    lax.fori_loop(0, _DROWS // _GRP, grp, 0)

    brow0 = (c * NS + s) * _BROWS
    pltpu.sync_copy(batch2d.at[pl.ds(brow0, _BROWS)], idx_v.at[pl.ds(0, _BROWS)])
    bs = [pltpu.async_copy(ones_v, cnt_s.at[idx_v.at[j]], sem, add=True)
          for j in range(_BROWS)]
    for d in bs:
        d.wait()

    plsc.subcore_barrier()
    pltpu.sync_copy(deg_s.at[pl.ds(s * _NT, _NT)], degp.at[c, pl.ds(s * _NT, _NT)])

    @pl.when(s == 0)
    def _():
        pltpu.sync_copy(cnt_s, cntp.at[c])


@functools.partial(
    pl.kernel,
    out_type=jax.ShapeDtypeStruct((NC, NPAD, 16), jnp.float32),
    mesh=_mesh(),
    scratch_types=[
        pltpu.VMEM_SHARED((NPAD, 16), jnp.float32),
        pltpu.VMEM((_GRP, 128), jnp.int32),
        pltpu.VMEM((_GRP, 128), jnp.int32),
        pltpu.VMEM((_GRP, 128, 16), jnp.float32),
        pltpu.VMEM((128, 16), jnp.float32),
        pltpu.SemaphoreType.DMA,
        pltpu.SemaphoreType.DMA,
        pltpu.SemaphoreType.DMA,
    ],
)
def _edge(srcA, dst2d, gflat, S, acc_s, src_v, dst_v, rows_v, zer_v,
          gsem, ssem, stsem):
    c = lax.axis_index("c")
    s = lax.axis_index("s")
    _fill(zer_v, 0.0)
    _zero_rows(acc_s, s * _NT, _NT // 128, zer_v, stsem)
    plsc.subcore_barrier()

    row0 = s * _ETILE

    def grp(gi, _):
        r = row0 + gi * _GRP
        d1 = pltpu.async_copy(srcA.at[c, pl.ds(r, _GRP)], src_v, stsem)
        d2 = pltpu.async_copy(dst2d.at[pl.ds(r, _GRP)], dst_v, stsem)
        d1.wait()
        d2.wait()
        gs = [pltpu.async_copy(gflat.at[src_v.at[j]], rows_v.at[j], gsem)
              for j in range(_GRP)]
        for d in gs:
            d.wait()
        ss = [pltpu.async_copy(rows_v.at[j], acc_s.at[dst_v.at[j]], ssem,
                               add=True) for j in range(_GRP)]
        for d in ss:
            d.wait()
        return 0
    lax.fori_loop(0, _NGRP, grp, 0)

    plsc.subcore_barrier()
    pltpu.sync_copy(acc_s.at[pl.ds(s * _NT, _NT)], S.at[c, pl.ds(s * _NT, _NT)])


@functools.partial(
    pl.kernel,
    out_type=jax.ShapeDtypeStruct((NC, GP, 16), jnp.float32),
    mesh=_mesh(),
    scratch_types=[
        pltpu.VMEM_SHARED((GP, 16), jnp.float32),
        pltpu.VMEM((NPAD // 128 // NS, 128), jnp.int32),
        pltpu.VMEM((_NT, 16), jnp.float32),
        pltpu.VMEM((128, 16), jnp.float32),
        pltpu.SemaphoreType.DMA,
    ],
)
def _pool(h3, batch2d, P, acc_s, idx_v, rows_v, zer_v, sem):
    c = lax.axis_index("c")
    s = lax.axis_index("s")
    _fill(zer_v, 0.0)
    pltpu.sync_copy(zer_v, acc_s.at[pl.ds(s * 128, 128)])

    @pl.when(s == 0)
    def _():
        pltpu.sync_copy(zer_v.at[pl.ds(0, 8)], acc_s.at[pl.ds(G, 8)])

    plsc.subcore_barrier()
    nrows = NPAD // 128 // NS
    pltpu.sync_copy(batch2d.at[pl.ds(s * nrows, nrows)], idx_v)
    pltpu.sync_copy(h3.at[c, pl.ds(s * _NT, _NT)], rows_v)
    ss = [pltpu.async_copy(rows_v.at[pl.ds(j * 128, 128)],
                           acc_s.at[idx_v.at[j]], sem, add=True)
          for j in range(nrows)]
    for d in ss:
        d.wait()
    plsc.subcore_barrier()
    pltpu.sync_copy(acc_s.at[pl.ds(s * 128, 128)], P.at[c, pl.ds(s * 128, 128)])

    @pl.when(s == 0)
    def _():
        pltpu.sync_copy(acc_s.at[pl.ds(G, 8)], P.at[c, pl.ds(G, 8)])


_B = 2048
_NBLK = NPAD // _B


def _scale1_body(x_ref, degp_ref, w_ref, dis_ref, g_ref):
    dp = degp_ref[...]
    dis = lax.rsqrt(dp[0] + dp[1] + 1.0)
    dis_ref[...] = dis
    hw = jnp.dot(x_ref[...], w_ref[...], preferred_element_type=jnp.float32)
    g_ref[...] = jnp.stack([hw[:, :16] * dis, hw[:, 16:] * dis], axis=0)


_scale1 = pl.pallas_call(
    _scale1_body,
    grid=(_NBLK,),
    in_specs=[pl.BlockSpec((_B, 32), lambda i: (i, 0)),
              pl.BlockSpec((NC, _B, 16), lambda i: (0, i, 0)),
              pl.BlockSpec((32, HID), lambda i: (0, 0))],
    out_specs=[pl.BlockSpec((_B, 16), lambda i: (i, 0)),
               pl.BlockSpec((NC, _B, 16), lambda i: (0, i, 0))],
    out_shape=[jax.ShapeDtypeStruct((NPAD, 16), jnp.float32),
               jax.ShapeDtypeStruct((NC, NPAD, 16), jnp.float32)],
)


def _scale_mid_body(S_ref, gp_ref, dis_ref, w_ref, b_ref, g_ref):
    dis = dis_ref[...]
    Sb = S_ref[...]
    gb = gp_ref[...]
    h = jnp.concatenate([dis * (Sb[0] + gb[0]), dis * (Sb[1] + gb[1])], axis=1)
    h = jnp.maximum(h + b_ref[...][0:1, :], 0.0)
    hw = jnp.dot(h, w_ref[...], preferred_element_type=jnp.float32)
    g_ref[...] = jnp.stack([hw[:, :16] * dis, hw[:, 16:] * dis], axis=0)


_scale_mid = pl.pallas_call(
    _scale_mid_body,
    grid=(_NBLK,),
    in_specs=[pl.BlockSpec((NC, _B, 16), lambda i: (0, i, 0)),
              pl.BlockSpec((NC, _B, 16), lambda i: (0, i, 0)),
              pl.BlockSpec((_B, 16), lambda i: (i, 0)),
              pl.BlockSpec((HID, HID), lambda i: (0, 0)),
              pl.BlockSpec((8, HID), lambda i: (0, 0))],
    out_specs=pl.BlockSpec((NC, _B, 16), lambda i: (0, i, 0)),
    out_shape=jax.ShapeDtypeStruct((NC, NPAD, 16), jnp.float32),
)


def _h3_body(S_ref, gp_ref, dis_ref, b_ref, h3_ref):
    dis = dis_ref[...]
    Sb = S_ref[...]
    gb = gp_ref[...]
    b = b_ref[...]
    h3_ref[...] = jnp.stack([dis * (Sb[0] + gb[0]) + b[0:1, :16],
                             dis * (Sb[1] + gb[1]) + b[0:1, 16:]], axis=0)


_h3k = pl.pallas_call(
    _h3_body,
    grid=(_NBLK,),
    in_specs=[pl.BlockSpec((NC, _B, 16), lambda i: (0, i, 0)),
              pl.BlockSpec((NC, _B, 16), lambda i: (0, i, 0)),
              pl.BlockSpec((_B, 16), lambda i: (i, 0)),
              pl.BlockSpec((8, OUT), lambda i: (0, 0))],
    out_specs=pl.BlockSpec((NC, _B, 16), lambda i: (0, i, 0)),
    out_shape=jax.ShapeDtypeStruct((NC, NPAD, 16), jnp.float32),
)


def _final_body(P_ref, cnt_ref, out_ref):
    P = P_ref[...]
    cnt = cnt_ref[...]
    c = jnp.maximum(cnt[0, :G, :] + cnt[1, :G, :], 1.0)
    out_ref[...] = jnp.concatenate([P[0, :G, :] / c, P[1, :G, :] / c], axis=1)


_final = pl.pallas_call(
    _final_body,
    out_shape=jax.ShapeDtypeStruct((G, OUT), jnp.float32),
)


def kernel(x, edge_index, batch, W1, b1, W2, b2, W3, b3):
    src = edge_index[0]
    dst = edge_index[1]
    src_p = jnp.pad(src, (0, EPAD - E))
    dst_p = jnp.pad(dst, (0, EPAD - E), constant_values=N)
    srcA = jnp.stack([src_p, src_p + NPAD]).reshape(NC, EROWS, 128)
    dst2d = dst_p.reshape(EROWS, 128)
    batch2d = jnp.pad(batch, (0, NPAD - N), constant_values=G).reshape(
        NPAD // 128, 128)
    x_p = jnp.pad(x, ((0, NPAD - N), (0, 32 - IN_DIM)))
    W1p = jnp.pad(W1, ((0, 32 - IN_DIM), (0, 0)))
    b1b = jnp.broadcast_to(b1[None, :], (8, HID))
    b2b = jnp.broadcast_to(b2[None, :], (8, HID))
    b3b = jnp.broadcast_to(b3[None, :], (8, OUT))

    degp, cntp = _prep(dst2d, batch2d)
    dis16, g1 = _scale1(x_p, degp, W1p)
    S1 = _edge(srcA, dst2d, g1.reshape(NC * NPAD, 16))
    g2 = _scale_mid(S1, g1, dis16, W2, b1b)
    S2 = _edge(srcA, dst2d, g2.reshape(NC * NPAD, 16))
    g3 = _scale_mid(S2, g2, dis16, W3, b2b)
    S3 = _edge(srcA, dst2d, g3.reshape(NC * NPAD, 16))
    h3 = _h3k(S3, g3, dis16, b3b)
    P = _pool(h3, batch2d)
    return _final(P, cntp)


# speedup vs baseline: 23.6326x; 23.6326x over previous
import functools

import jax
import jax.numpy as jnp
from jax import lax
from jax.experimental import pallas as pl
from jax.experimental.pallas import tpu as pltpu
from jax.experimental.pallas import tpu_sc as plsc

N = 100000
E = 1600000
IN_DIM = 26
HID = 32
OUT = 32
G = 2048

NC, NS = 2, 16
NPAD = 102400
EPAD = 1605632
EROWS = EPAD // 128
GP = G + 8

_ETILE = EROWS // NS
_GRP = 16
_NGRP = _ETILE // _GRP
_SB = 8
_DROWS = EROWS // (NC * NS)
_DGRP = 56
_NT = NPAD // NS
NACC = 100008
_WB = 6256
_WBL = NACC - 15 * _WB


def _mesh():
    return plsc.VectorSubcoreMesh(core_axis_name="c", subcore_axis_name="s",
                                  num_cores=NC, num_subcores=NS)


def _fill(ref, val):
    def body(i, _):
        ref[i] = jnp.full((16,), val, jnp.float32)
        return 0
    lax.fori_loop(0, 128, body, 0)


def _zero_rows(dst, base, n, zer_v, sem):
    ds = [pltpu.async_copy(zer_v, dst.at[pl.ds(base + i * 128, 128)], sem)
          for i in range(n)]
    for d in ds:
        d.wait()


@functools.partial(
    pl.kernel,
    out_type=[jax.ShapeDtypeStruct((NC, NPAD, 16), jnp.float32),
              jax.ShapeDtypeStruct((NC, GP, 16), jnp.float32)],
    mesh=_mesh(),
    compiler_params=pltpu.CompilerParams(use_tc_tiling_on_sc=False),
    scratch_types=[
        pltpu.VMEM_SHARED((NPAD, 16), jnp.float32),
        pltpu.VMEM_SHARED((GP, 16), jnp.float32),
        pltpu.VMEM((_DGRP, 128), jnp.int32),
        pltpu.VMEM((128, 16), jnp.float32),
        pltpu.VMEM((128, 16), jnp.float32),
        pltpu.SemaphoreType.DMA,
    ],
)
def _prep(dst2d, batch2d, degp, cntp, deg_s, cnt_s, idx_v, ones_v, zer_v, sem):
    c = lax.axis_index("c")
    s = lax.axis_index("s")
    _fill(ones_v, 1.0)
    _fill(zer_v, 0.0)
    _zero_rows(deg_s, s * _NT, _NT // 128, zer_v, sem)

    @pl.when(s == 0)
    def _():
        _zero_rows(cnt_s, 0, G // 128, zer_v, sem)
        pltpu.sync_copy(zer_v.at[pl.ds(0, 8)], cnt_s.at[pl.ds(G, 8)])

    plsc.subcore_barrier()

    row0 = c * (EROWS // NC) + s * _DROWS

    def grp(gi, _):
        pltpu.sync_copy(dst2d.at[pl.ds(row0 + gi * _DGRP, _DGRP)], idx_v)
        ds = [pltpu.async_copy(ones_v, deg_s.at[idx_v.at[j]],
                               sem, add=True) for j in range(_DGRP)]
        for d in ds:
            d.wait()
        return 0
    lax.fori_loop(0, _DROWS // _DGRP, grp, 0)

    w = c * NS + s
    pltpu.sync_copy(batch2d.at[pl.ds(w * 24, 24)], idx_v.at[pl.ds(0, 24)])
    bs = [pltpu.async_copy(ones_v, cnt_s.at[idx_v.at[j]], sem, add=True)
          for j in range(24)]
    for d in bs:
        d.wait()

    @pl.when(w < 4)
    def _():
        pltpu.sync_copy(batch2d.at[pl.ds(768 + w * 8, 8)], idx_v.at[pl.ds(0, 8)])
        bs2 = [pltpu.async_copy(ones_v, cnt_s.at[idx_v.at[j]], sem, add=True)
               for j in range(8)]
        for d in bs2:
            d.wait()

    plsc.subcore_barrier()
    pltpu.sync_copy(deg_s.at[pl.ds(s * _NT, _NT)], degp.at[c, pl.ds(s * _NT, _NT)])

    @pl.when(s == 0)
    def _():
        pltpu.sync_copy(cnt_s, cntp.at[c])


@functools.partial(
    pl.kernel,
    out_type=jax.ShapeDtypeStruct((NC, NPAD, 16), jnp.float32),
    mesh=_mesh(),
    compiler_params=pltpu.CompilerParams(use_tc_tiling_on_sc=False),
    scratch_types=[
        pltpu.VMEM_SHARED((NACC, 16), jnp.float32),
        pltpu.VMEM((_GRP, 128), jnp.int32),
        pltpu.VMEM((_GRP, 128), jnp.int32),
        pltpu.VMEM((_SB * 128, 16), jnp.float32),
        pltpu.VMEM((128, 16), jnp.float32),
        pltpu.SemaphoreType.DMA,
        pltpu.SemaphoreType.DMA,
        pltpu.SemaphoreType.DMA,
    ],
)
def _edge(srcA, dst2d, gflat, S, acc_s, src_v, dst_v, rows_v, zer_v,
          gsem, ssem, stsem):
    c = lax.axis_index("c")
    s = lax.axis_index("s")
    _fill(zer_v, 0.0)
    _zero_rows(acc_s, s * _WB, 48, zer_v, stsem)

    @pl.when(s < 15)
    def _():
        pltpu.sync_copy(zer_v.at[pl.ds(0, _WB - 48 * 128)],
                        acc_s.at[pl.ds(s * _WB + 48 * 128, _WB - 48 * 128)])

    @pl.when(s == 15)
    def _():
        pltpu.sync_copy(zer_v.at[pl.ds(0, _WBL - 48 * 128)],
                        acc_s.at[pl.ds(s * _WB + 48 * 128, _WBL - 48 * 128)])

    plsc.subcore_barrier()

    row0 = s * _ETILE

    def grp(gi, _):
        r = row0 + gi * _GRP
        d1 = pltpu.async_copy(srcA.at[c, pl.ds(r, _GRP)], src_v, stsem)
        d2 = pltpu.async_copy(dst2d.at[pl.ds(r, _GRP)], dst_v, stsem)
        d1.wait()
        d2.wait()
        for h in range(_GRP // _SB):
            gs = [pltpu.async_copy(gflat.at[src_v.at[h * _SB + j]],
                                   rows_v.at[pl.ds(j * 128, 128)], gsem)
                  for j in range(_SB)]
            for d in gs:
                d.wait()
            ss = [pltpu.async_copy(rows_v.at[pl.ds(j * 128, 128)],
                                   acc_s.at[dst_v.at[h * _SB + j]], ssem,
                                   add=True) for j in range(_SB)]
            for d in ss:
                d.wait()
        return 0
    lax.fori_loop(0, _NGRP, grp, 0)

    plsc.subcore_barrier()

    @pl.when(s < 15)
    def _():
        pltpu.sync_copy(acc_s.at[pl.ds(s * _WB, _WB)],
                        S.at[c, pl.ds(s * _WB, _WB)])

    @pl.when(s == 15)
    def _():
        pltpu.sync_copy(acc_s.at[pl.ds(s * _WB, _WBL)],
                        S.at[c, pl.ds(s * _WB, _WBL)])


@functools.partial(
    pl.kernel,
    out_type=jax.ShapeDtypeStruct((NC, GP, 16), jnp.float32),
    mesh=_mesh(),
    compiler_params=pltpu.CompilerParams(use_tc_tiling_on_sc=False),
    scratch_types=[
        pltpu.VMEM_SHARED((GP, 16), jnp.float32),
        pltpu.VMEM((48, 128), jnp.int32),
        pltpu.VMEM((48 * 128, 16), jnp.float32),
        pltpu.VMEM((128, 16), jnp.float32),
        pltpu.SemaphoreType.DMA,
    ],
)
def _pool(h3, batch2d, P, acc_s, idx_v, rows_v, zer_v, sem):
    c = lax.axis_index("c")
    s = lax.axis_index("s")
    _fill(zer_v, 0.0)
    pltpu.sync_copy(zer_v, acc_s.at[pl.ds(s * 128, 128)])

    @pl.when(s == 0)
    def _():
        pltpu.sync_copy(zer_v.at[pl.ds(0, 8)], acc_s.at[pl.ds(G, 8)])

    plsc.subcore_barrier()
    pltpu.sync_copy(batch2d.at[pl.ds(s * 48, 48)], idx_v)
    pltpu.sync_copy(h3.at[c, pl.ds(s * 48 * 128, 48 * 128)], rows_v)
    ss = [pltpu.async_copy(rows_v.at[pl.ds(j * 128, 128)],
                           acc_s.at[idx_v.at[j]], sem, add=True)
          for j in range(48)]
    for d in ss:
        d.wait()

    @pl.when(s < 4)
    def _():
        pltpu.sync_copy(batch2d.at[pl.ds(768 + s * 8, 8)], idx_v.at[pl.ds(0, 8)])
        pltpu.sync_copy(h3.at[c, pl.ds(768 * 128 + s * 1024, 1024)],
                        rows_v.at[pl.ds(0, 1024)])
        ss2 = [pltpu.async_copy(rows_v.at[pl.ds(j * 128, 128)],
                                acc_s.at[idx_v.at[j]], sem, add=True)
               for j in range(8)]
        for d in ss2:
            d.wait()
    plsc.subcore_barrier()
    pltpu.sync_copy(acc_s.at[pl.ds(s * 128, 128)], P.at[c, pl.ds(s * 128, 128)])

    @pl.when(s == 0)
    def _():
        pltpu.sync_copy(acc_s.at[pl.ds(G, 8)], P.at[c, pl.ds(G, 8)])


_B = 2048
_NBLK = NPAD // _B


def _scale1_body(x_ref, degp_ref, w_ref, dis_ref, g_ref):
    dp = degp_ref[...]
    dis = lax.rsqrt(dp[0] + dp[1] + 1.0)
    dis_ref[...] = dis
    hw = jnp.dot(x_ref[...], w_ref[...], preferred_element_type=jnp.float32)
    g_ref[...] = jnp.stack([hw[:, :16] * dis, hw[:, 16:] * dis], axis=0)


_scale1 = pl.pallas_call(
    _scale1_body,
    grid=(_NBLK,),
    in_specs=[pl.BlockSpec((_B, 32), lambda i: (i, 0)),
              pl.BlockSpec((NC, _B, 16), lambda i: (0, i, 0)),
              pl.BlockSpec((32, HID), lambda i: (0, 0))],
    out_specs=[pl.BlockSpec((_B, 16), lambda i: (i, 0)),
               pl.BlockSpec((NC, _B, 16), lambda i: (0, i, 0))],
    out_shape=[jax.ShapeDtypeStruct((NPAD, 16), jnp.float32),
               jax.ShapeDtypeStruct((NC, NPAD, 16), jnp.float32)],
)


def _scale_mid_body(S_ref, gp_ref, dis_ref, w_ref, b_ref, g_ref):
    dis = dis_ref[...]
    Sb = S_ref[...]
    gb = gp_ref[...]
    h = jnp.concatenate([dis * (Sb[0] + gb[0]), dis * (Sb[1] + gb[1])], axis=1)
    h = jnp.maximum(h + b_ref[...][0:1, :], 0.0)
    hw = jnp.dot(h, w_ref[...], preferred_element_type=jnp.float32)
    g_ref[...] = jnp.stack([hw[:, :16] * dis, hw[:, 16:] * dis], axis=0)


_scale_mid = pl.pallas_call(
    _scale_mid_body,
    grid=(_NBLK,),
    in_specs=[pl.BlockSpec((NC, _B, 16), lambda i: (0, i, 0)),
              pl.BlockSpec((NC, _B, 16), lambda i: (0, i, 0)),
              pl.BlockSpec((_B, 16), lambda i: (i, 0)),
              pl.BlockSpec((HID, HID), lambda i: (0, 0)),
              pl.BlockSpec((8, HID), lambda i: (0, 0))],
    out_specs=pl.BlockSpec((NC, _B, 16), lambda i: (0, i, 0)),
    out_shape=jax.ShapeDtypeStruct((NC, NPAD, 16), jnp.float32),
)


def _h3_body(S_ref, gp_ref, dis_ref, b_ref, h3_ref):
    dis = dis_ref[...]
    Sb = S_ref[...]
    gb = gp_ref[...]
    b = b_ref[...]
    h3_ref[...] = jnp.stack([dis * (Sb[0] + gb[0]) + b[0:1, :16],
                             dis * (Sb[1] + gb[1]) + b[0:1, 16:]], axis=0)


_h3k = pl.pallas_call(
    _h3_body,
    grid=(_NBLK,),
    in_specs=[pl.BlockSpec((NC, _B, 16), lambda i: (0, i, 0)),
              pl.BlockSpec((NC, _B, 16), lambda i: (0, i, 0)),
              pl.BlockSpec((_B, 16), lambda i: (i, 0)),
              pl.BlockSpec((8, OUT), lambda i: (0, 0))],
    out_specs=pl.BlockSpec((NC, _B, 16), lambda i: (0, i, 0)),
    out_shape=jax.ShapeDtypeStruct((NC, NPAD, 16), jnp.float32),
)


def _final_body(P_ref, cnt_ref, out_ref):
    P = P_ref[...]
    cnt = cnt_ref[...]
    c = jnp.maximum(cnt[0, :G, :] + cnt[1, :G, :], 1.0)
    out_ref[...] = jnp.concatenate([P[0, :G, :] / c, P[1, :G, :] / c], axis=1)


_final = pl.pallas_call(
    _final_body,
    out_shape=jax.ShapeDtypeStruct((G, OUT), jnp.float32),
)


def kernel(x, edge_index, batch, W1, b1, W2, b2, W3, b3):
    src = edge_index[0]
    dst = edge_index[1]
    src_p = jnp.pad(src, (0, EPAD - E))
    dst_p = jnp.pad(dst, (0, EPAD - E), constant_values=N)
    srcA = jnp.stack([src_p, src_p + NPAD]).reshape(NC, EROWS, 128)
    dst2d = dst_p.reshape(EROWS, 128)
    batch2d = jnp.pad(batch, (0, NPAD - N), constant_values=G).reshape(
        NPAD // 128, 128)
    x_p = jnp.pad(x, ((0, NPAD - N), (0, 32 - IN_DIM)))
    W1p = jnp.pad(W1, ((0, 32 - IN_DIM), (0, 0)))
    b1b = jnp.broadcast_to(b1[None, :], (8, HID))
    b2b = jnp.broadcast_to(b2[None, :], (8, HID))
    b3b = jnp.broadcast_to(b3[None, :], (8, OUT))

    degp, cntp = _prep(dst2d, batch2d)
    dis16, g1 = _scale1(x_p, degp, W1p)
    S1 = _edge(srcA, dst2d, g1.reshape(NC * NPAD, 16))
    g2 = _scale_mid(S1, g1, dis16, W2, b1b)
    S2 = _edge(srcA, dst2d, g2.reshape(NC * NPAD, 16))
    g3 = _scale_mid(S2, g2, dis16, W3, b2b)
    S3 = _edge(srcA, dst2d, g3.reshape(NC * NPAD, 16))
    h3 = _h3k(S3, g3, dis16, b3b)
    P = _pool(h3, batch2d)
    return _final(P, cntp)
